# baseline (speedup 1.0000x reference)
import jax
import jax.numpy as jnp
from jax import lax
from jax.experimental import pallas as pl
from jax.experimental.pallas import tpu as pltpu

B, V = 16384, 1000
GRID = 8
CB = B // GRID


def _body(ia_ref, ib_ref, ig_ref, a_ref, b_ref, g_ref, o_ref):
    sub = lax.broadcasted_iota(jnp.int32, (8, CB), 0)
    av = jnp.sum(jnp.where(sub == ia_ref[0] % 8, a_ref[...], 0.0), axis=0)
    bv = jnp.sum(jnp.where(sub == ib_ref[0] % 8, b_ref[...], 0.0), axis=0)
    gv = jnp.sum(jnp.where(sub == ig_ref[0] % 8, g_ref[...], 0.0), axis=0)
    o_ref[...] = jnp.maximum(av + bv - gv, 0.0)


_grid_spec = pltpu.PrefetchScalarGridSpec(
    num_scalar_prefetch=3,
    grid=(GRID,),
    in_specs=[
        pl.BlockSpec((8, CB), lambda i, ia, ib, ig: (ia[0] // 8, i)),
        pl.BlockSpec((8, CB), lambda i, ia, ib, ig: (ib[0] // 8, i)),
        pl.BlockSpec((8, CB), lambda i, ia, ib, ig: (ig[0] // 8, i)),
    ],
    out_specs=pl.BlockSpec((CB,), lambda i, ia, ib, ig: (i,)),
)

_call = pl.pallas_call(
    _body,
    grid_spec=_grid_spec,
    out_shape=jax.ShapeDtypeStruct((B,), jnp.float32),
)


def kernel(log_y_alpha, log_y_beta, log_y_gamma, alpha_index, beta_index, gamma_index):
    ia = jnp.full((1,), alpha_index, dtype=jnp.int32)
    ib = jnp.full((1,), beta_index, dtype=jnp.int32)
    ig = jnp.full((1,), gamma_index, dtype=jnp.int32)
    return _call(ia, ib, ig, log_y_alpha.T, log_y_beta.T, log_y_gamma.T)


# speedup vs baseline: 1.5472x; 1.5472x over previous
import jax
import jax.numpy as jnp
from jax.experimental import pallas as pl
from jax.experimental.pallas import tpu as pltpu

B, V = 16384, 1000


def _body(cols_ref, a_hbm, b_hbm, g_hbm, o_ref,
          a_v, b_v, g_v, sem_a, sem_b, sem_g):
    cp_a = pltpu.make_async_copy(a_hbm.at[pl.ds(cols_ref[0], 1)], a_v, sem_a)
    cp_b = pltpu.make_async_copy(b_hbm.at[pl.ds(cols_ref[1], 1)], b_v, sem_b)
    cp_g = pltpu.make_async_copy(g_hbm.at[pl.ds(cols_ref[2], 1)], g_v, sem_g)
    cp_a.start()
    cp_b.start()
    cp_g.start()
    cp_a.wait()
    cp_b.wait()
    cp_g.wait()
    o_ref[...] = jnp.maximum(a_v[0, :] + b_v[0, :] - g_v[0, :], 0.0)


_call = pl.pallas_call(
    _body,
    in_specs=[
        pl.BlockSpec(memory_space=pltpu.MemorySpace.SMEM),
        pl.BlockSpec(memory_space=pltpu.MemorySpace.HBM),
        pl.BlockSpec(memory_space=pltpu.MemorySpace.HBM),
        pl.BlockSpec(memory_space=pltpu.MemorySpace.HBM),
    ],
    out_specs=pl.BlockSpec(memory_space=pltpu.MemorySpace.VMEM),
    out_shape=jax.ShapeDtypeStruct((B,), jnp.float32),
    scratch_shapes=[
        pltpu.VMEM((1, B), jnp.float32),
        pltpu.VMEM((1, B), jnp.float32),
        pltpu.VMEM((1, B), jnp.float32),
        pltpu.SemaphoreType.DMA,
        pltpu.SemaphoreType.DMA,
        pltpu.SemaphoreType.DMA,
    ],
)


def kernel(log_y_alpha, log_y_beta, log_y_gamma, alpha_index, beta_index, gamma_index):
    cols = jnp.stack([
        jnp.asarray(alpha_index, dtype=jnp.int32),
        jnp.asarray(beta_index, dtype=jnp.int32),
        jnp.asarray(gamma_index, dtype=jnp.int32),
    ])
    return _call(cols, log_y_alpha.T, log_y_beta.T, log_y_gamma.T)


# speedup vs baseline: 2.0537x; 1.3274x over previous
import jax
import jax.numpy as jnp
from jax.experimental import pallas as pl
from jax.experimental.pallas import tpu as pltpu

B, V = 16384, 1000


def _body(ia_ref, ib_ref, ig_ref, a_hbm, b_hbm, g_hbm, o_ref,
          a_v, b_v, g_v, sem_a, sem_b, sem_g):
    cp_a = pltpu.make_async_copy(a_hbm.at[pl.ds(ia_ref[0], 1)], a_v, sem_a)
    cp_b = pltpu.make_async_copy(b_hbm.at[pl.ds(ib_ref[0], 1)], b_v, sem_b)
    cp_g = pltpu.make_async_copy(g_hbm.at[pl.ds(ig_ref[0], 1)], g_v, sem_g)
    cp_a.start(priority=0)
    cp_b.start(priority=1)
    cp_g.start(priority=0)
    cp_a.wait()
    cp_b.wait()
    cp_g.wait()
    o_ref[...] = jnp.maximum(a_v[0, :] + b_v[0, :] - g_v[0, :], 0.0)


_call = pl.pallas_call(
    _body,
    in_specs=[
        pl.BlockSpec(memory_space=pltpu.MemorySpace.SMEM),
        pl.BlockSpec(memory_space=pltpu.MemorySpace.SMEM),
        pl.BlockSpec(memory_space=pltpu.MemorySpace.SMEM),
        pl.BlockSpec(memory_space=pltpu.MemorySpace.HBM),
        pl.BlockSpec(memory_space=pltpu.MemorySpace.HBM),
        pl.BlockSpec(memory_space=pltpu.MemorySpace.HBM),
    ],
    out_specs=pl.BlockSpec(memory_space=pltpu.MemorySpace.VMEM),
    out_shape=jax.ShapeDtypeStruct((B,), jnp.float32),
    scratch_shapes=[
        pltpu.VMEM((1, B), jnp.float32),
        pltpu.VMEM((1, B), jnp.float32),
        pltpu.VMEM((1, B), jnp.float32),
        pltpu.SemaphoreType.DMA,
        pltpu.SemaphoreType.DMA,
        pltpu.SemaphoreType.DMA,
    ],
)


def kernel(log_y_alpha, log_y_beta, log_y_gamma, alpha_index, beta_index, gamma_index):
    ia = jnp.full((1,), alpha_index, dtype=jnp.int32)
    ib = jnp.full((1,), beta_index, dtype=jnp.int32)
    ig = jnp.full((1,), gamma_index, dtype=jnp.int32)
    return _call(ia, ib, ig, log_y_alpha.T, log_y_beta.T, log_y_gamma.T)


# speedup vs baseline: 2.1194x; 1.0320x over previous
import jax
import jax.numpy as jnp
from jax.experimental import pallas as pl
from jax.experimental.pallas import tpu as pltpu

B, V = 16384, 1000
H = B // 2


def _body(ia_ref, ib_ref, ig_ref, a_hbm, b_hbm, g_hbm, o_ref,
          a_v, b_v, g_v, sem0, sem1):
    cps = []
    for hbm, idx_ref, v in ((a_hbm, ia_ref, a_v), (b_hbm, ib_ref, b_v),
                            (g_hbm, ig_ref, g_v)):
        row = hbm.at[pl.ds(idx_ref[0], 1)]
        cp0 = pltpu.make_async_copy(row.at[:, pl.ds(0, H)], v.at[:, pl.ds(0, H)], sem0)
        cp1 = pltpu.make_async_copy(row.at[:, pl.ds(H, H)], v.at[:, pl.ds(H, H)], sem1)
        cp0.start(priority=0)
        cp1.start(priority=1)
        cps += [cp0, cp1]
    for cp in cps:
        cp.wait()
    o_ref[...] = jnp.maximum(a_v[0, :] + b_v[0, :] - g_v[0, :], 0.0)


_call = pl.pallas_call(
    _body,
    in_specs=[
        pl.BlockSpec(memory_space=pltpu.MemorySpace.SMEM),
        pl.BlockSpec(memory_space=pltpu.MemorySpace.SMEM),
        pl.BlockSpec(memory_space=pltpu.MemorySpace.SMEM),
        pl.BlockSpec(memory_space=pltpu.MemorySpace.HBM),
        pl.BlockSpec(memory_space=pltpu.MemorySpace.HBM),
        pl.BlockSpec(memory_space=pltpu.MemorySpace.HBM),
    ],
    out_specs=pl.BlockSpec(memory_space=pltpu.MemorySpace.VMEM),
    out_shape=jax.ShapeDtypeStruct((B,), jnp.float32),
    scratch_shapes=[
        pltpu.VMEM((1, B), jnp.float32),
        pltpu.VMEM((1, B), jnp.float32),
        pltpu.VMEM((1, B), jnp.float32),
        pltpu.SemaphoreType.DMA,
        pltpu.SemaphoreType.DMA,
    ],
)


def kernel(log_y_alpha, log_y_beta, log_y_gamma, alpha_index, beta_index, gamma_index):
    ia = jnp.full((1,), alpha_index, dtype=jnp.int32)
    ib = jnp.full((1,), beta_index, dtype=jnp.int32)
    ig = jnp.full((1,), gamma_index, dtype=jnp.int32)
    return _call(ia, ib, ig, log_y_alpha.T, log_y_beta.T, log_y_gamma.T)
